# baseline (speedup 1.0000x reference)
import jax
import jax.numpy as jnp
import numpy as np
from jax import lax
from jax.experimental import pallas as pl
from jax.experimental.pallas import tpu as pltpu
from jax.experimental.pallas import tpu_sc as plsc

D_MODEL = 64
MAX_LEN = 256
SEQ = 200

NW = 32
GRP = 128
BQ = 256
NQ = 4
NLANE = 16
SCALE = float(np.sqrt(np.float32(D_MODEL)))


def _pe_np():
    pos = np.arange(MAX_LEN)[:, np.newaxis]
    i = np.arange(D_MODEL)[np.newaxis, :]
    angle_rates = 1 / np.power(10000, 2 * (i // 2) / np.float32(D_MODEL))
    angle_rads = pos * angle_rates
    pe = np.zeros((MAX_LEN, D_MODEL), dtype=np.float32)
    pe[:, 0::2] = np.sin(angle_rads[:, 0::2])
    pe[:, 1::2] = np.cos(angle_rads[:, 1::2])
    return pe[:SEQ]


NBUF_G = 3
NBUF_O = 2


def _sc_body(table_hbm, idx_hbm, pe_hbm, out_hbm, idx_v, pe_v,
             bufs, obufs, sems_g, sems_w):
    wid = lax.axis_index("s") * 2 + lax.axis_index("c")
    units_per_w = (SEQ * NQ) // NW
    grps_per_unit = BQ // GRP

    pltpu.sync_copy(idx_hbm.at[wid], idx_v)
    pltpu.sync_copy(pe_hbm, pe_v)

    drow = [lax.iota(jnp.int32, NLANE) + db * NLANE for db in range(D_MODEL // NLANE)]

    def fire_gather(i):
        slot = i % NBUF_G
        for g in range(grps_per_unit):
            pltpu.async_copy(
                table_hbm.at[idx_v.at[i * grps_per_unit + g]],
                bufs[slot].at[pl.ds(g * GRP, GRP)],
                sems_g[slot],
            )

    def wait_gather(i):
        slot = i % NBUF_G
        for g in range(grps_per_unit):
            pltpu.make_async_copy(
                table_hbm.at[idx_v.at[g]],
                bufs[slot].at[pl.ds(g * GRP, GRP)],
                sems_g[slot],
            ).wait()

    def unit_sq(i):
        u = wid * units_per_w + i
        return u // NQ, lax.rem(u, NQ)

    def wait_write(i):
        s, q = unit_sq(i)
        pltpu.make_async_copy(
            obufs[i % NBUF_O],
            out_hbm.at[s, :, pl.ds(q * BQ, BQ)],
            sems_w[i % NBUF_O],
        ).wait()

    for i in range(min(NBUF_G, units_per_w)):
        fire_gather(i)

    for i in range(units_per_w):
        s, q = unit_sq(i)
        buf = bufs[i % NBUF_G]
        obuf = obufs[i % NBUF_O]

        with jax.named_scope("xp_gwait"):
            wait_gather(i)
        if i >= NBUF_O:
            with jax.named_scope("xp_wwait"):
                wait_write(i - NBUF_O)

        pe_s = [pe_v[s, pl.ds(db * NLANE, NLANE)] for db in range(D_MODEL // NLANE)]

        def j_body(j, carry2):
            col = jnp.full((NLANE,), j, dtype=jnp.int32)
            for db in range(D_MODEL // NLANE):
                v = buf[j, pl.ds(db * NLANE, NLANE)]
                plsc.store_scatter(obuf, [drow[db], col], v * SCALE + pe_s[db])
            return carry2

        with jax.named_scope("xp_compute"):
            lax.fori_loop(0, BQ, j_body, 0)

        pltpu.async_copy(obuf, out_hbm.at[s, :, pl.ds(q * BQ, BQ)], sems_w[i % NBUF_O])
        if i + NBUF_G < units_per_w:
            fire_gather(i + NBUF_G)

    for i in range(units_per_w - NBUF_O, units_per_w):
        wait_write(i)


def kernel(x, table):
    batch, seq = x.shape
    n = batch * seq
    xq = jnp.transpose(x).reshape(NW, n // (NW * GRP), GRP)
    pe = jnp.asarray(_pe_np())

    mesh = plsc.VectorSubcoreMesh(core_axis_name="c", subcore_axis_name="s")
    run = pl.kernel(
        _sc_body,
        mesh=mesh,
        out_type=jax.ShapeDtypeStruct((seq, D_MODEL, batch), jnp.float32),
        scratch_types=[
            pltpu.VMEM((n // (NW * GRP), GRP), jnp.int32),
            pltpu.VMEM((SEQ, D_MODEL), jnp.float32),
            [pltpu.VMEM((BQ, D_MODEL), jnp.float32) for _ in range(NBUF_G)],
            [pltpu.VMEM((D_MODEL, BQ), jnp.float32) for _ in range(NBUF_O)],
            [pltpu.SemaphoreType.DMA for _ in range(NBUF_G)],
            [pltpu.SemaphoreType.DMA for _ in range(NBUF_O)],
        ],
        compiler_params=pltpu.CompilerParams(
            use_tc_tiling_on_sc=False, needs_layout_passes=False),
    )
    out = run(table, xq, pe)
    return jnp.transpose(out, (2, 0, 1))


# speedup vs baseline: 1.1512x; 1.1512x over previous
import jax
import jax.numpy as jnp
import numpy as np
from jax import lax
from jax.experimental import pallas as pl
from jax.experimental.pallas import tpu as pltpu
from jax.experimental.pallas import tpu_sc as plsc

D_MODEL = 64
MAX_LEN = 256
SEQ = 200

NW = 32
GRP = 128
BQ = 256
NQ = 4
NLANE = 16
SCALE = float(np.sqrt(np.float32(D_MODEL)))


def _pe_np():
    pos = np.arange(MAX_LEN)[:, np.newaxis]
    i = np.arange(D_MODEL)[np.newaxis, :]
    angle_rates = 1 / np.power(10000, 2 * (i // 2) / np.float32(D_MODEL))
    angle_rads = pos * angle_rates
    pe = np.zeros((MAX_LEN, D_MODEL), dtype=np.float32)
    pe[:, 0::2] = np.sin(angle_rads[:, 0::2])
    pe[:, 1::2] = np.cos(angle_rads[:, 1::2])
    return pe[:SEQ]


NBUF_G = 3
NBUF_O = 2


def _sc_body(table_hbm, idx_hbm, pe_hbm, out_hbm, idx_v, pe_v,
             bufs, obufs, sems_g, sems_w):
    wid = lax.axis_index("s") * 2 + lax.axis_index("c")
    units_per_w = (SEQ * NQ) // NW
    grps_per_unit = BQ // GRP

    pltpu.sync_copy(idx_hbm.at[wid], idx_v)
    pltpu.sync_copy(pe_hbm, pe_v)

    drow = [lax.iota(jnp.int32, NLANE) + db * NLANE for db in range(D_MODEL // NLANE)]

    def fire_gather(i):
        slot = i % NBUF_G
        for g in range(grps_per_unit):
            pltpu.async_copy(
                table_hbm.at[idx_v.at[i * grps_per_unit + g]],
                bufs[slot].at[pl.ds(g * GRP, GRP)],
                sems_g[slot],
            )

    def wait_gather(i):
        slot = i % NBUF_G
        for g in range(grps_per_unit):
            pltpu.make_async_copy(
                table_hbm.at[idx_v.at[g]],
                bufs[slot].at[pl.ds(g * GRP, GRP)],
                sems_g[slot],
            ).wait()

    def unit_sq(i):
        u = wid * units_per_w + i
        return u // NQ, lax.rem(u, NQ)

    def wait_write(i):
        s, q = unit_sq(i)
        pltpu.make_async_copy(
            obufs[i % NBUF_O],
            out_hbm.at[s, :, pl.ds(q * BQ, BQ)],
            sems_w[i % NBUF_O],
        ).wait()

    for i in range(min(NBUF_G, units_per_w)):
        fire_gather(i)

    for i in range(units_per_w):
        s, q = unit_sq(i)
        buf = bufs[i % NBUF_G]
        obuf = obufs[i % NBUF_O]

        with jax.named_scope("xp_gwait"):
            wait_gather(i)
        if i >= NBUF_O:
            with jax.named_scope("xp_wwait"):
                wait_write(i - NBUF_O)

        pe_s = [pe_v[s, pl.ds(db * NLANE, NLANE)] for db in range(D_MODEL // NLANE)]

        def j_body(j):
            col = jnp.full((NLANE,), j, dtype=jnp.int32)
            for db in range(D_MODEL // NLANE):
                v = buf[j, pl.ds(db * NLANE, NLANE)]
                plsc.store_scatter(obuf, [drow[db], col], v * SCALE + pe_s[db])

        with jax.named_scope("xp_compute"):
            plsc.parallel_loop(0, BQ, unroll=4)(j_body)

        pltpu.async_copy(obuf, out_hbm.at[s, :, pl.ds(q * BQ, BQ)], sems_w[i % NBUF_O])
        if i + NBUF_G < units_per_w:
            fire_gather(i + NBUF_G)

    for i in range(units_per_w - NBUF_O, units_per_w):
        wait_write(i)


def kernel(x, table):
    batch, seq = x.shape
    n = batch * seq
    xq = jnp.transpose(x).reshape(NW, n // (NW * GRP), GRP)
    pe = jnp.asarray(_pe_np())

    mesh = plsc.VectorSubcoreMesh(core_axis_name="c", subcore_axis_name="s")
    run = pl.kernel(
        _sc_body,
        mesh=mesh,
        out_type=jax.ShapeDtypeStruct((seq, D_MODEL, batch), jnp.float32),
        scratch_types=[
            pltpu.VMEM((n // (NW * GRP), GRP), jnp.int32),
            pltpu.VMEM((SEQ, D_MODEL), jnp.float32),
            [pltpu.VMEM((BQ, D_MODEL), jnp.float32) for _ in range(NBUF_G)],
            [pltpu.VMEM((D_MODEL, BQ), jnp.float32) for _ in range(NBUF_O)],
            [pltpu.SemaphoreType.DMA for _ in range(NBUF_G)],
            [pltpu.SemaphoreType.DMA for _ in range(NBUF_O)],
        ],
        compiler_params=pltpu.CompilerParams(
            use_tc_tiling_on_sc=False, needs_layout_passes=False),
    )
    out = run(table, xq, pe)
    return jnp.transpose(out, (2, 0, 1))


# speedup vs baseline: 1.3822x; 1.2006x over previous
import jax
import jax.numpy as jnp
import numpy as np
from jax import lax
from jax.experimental import pallas as pl
from jax.experimental.pallas import tpu as pltpu
from jax.experimental.pallas import tpu_sc as plsc

D_MODEL = 64
MAX_LEN = 256
SEQ = 200

NW = 32
GRP = 128
BQ = 256
BQP = BQ + 1
NQ = 4
NLANE = 16
SCALE = float(np.sqrt(np.float32(D_MODEL)))


def _pe_np():
    pos = np.arange(MAX_LEN)[:, np.newaxis]
    i = np.arange(D_MODEL)[np.newaxis, :]
    angle_rates = 1 / np.power(10000, 2 * (i // 2) / np.float32(D_MODEL))
    angle_rads = pos * angle_rates
    pe = np.zeros((MAX_LEN, D_MODEL), dtype=np.float32)
    pe[:, 0::2] = np.sin(angle_rads[:, 0::2])
    pe[:, 1::2] = np.cos(angle_rads[:, 1::2])
    return pe[:SEQ]


NBUF_G = 3
NBUF_O = 2


def _sc_body(table_hbm, idx_hbm, pe_hbm, out_hbm, idx_v, pe_v,
             bufs, obufs, sems_g, sems_w):
    wid = lax.axis_index("s") * 2 + lax.axis_index("c")
    units_per_w = (SEQ * NQ) // NW
    grps_per_unit = BQ // GRP

    pltpu.sync_copy(idx_hbm.at[wid], idx_v)
    pltpu.sync_copy(pe_hbm, pe_v)

    drow = [lax.iota(jnp.int32, NLANE) + db * NLANE for db in range(D_MODEL // NLANE)]

    def fire_gather(i):
        slot = i % NBUF_G
        for g in range(grps_per_unit):
            pltpu.async_copy(
                table_hbm.at[idx_v.at[i * grps_per_unit + g]],
                bufs[slot].at[pl.ds(g * GRP, GRP)],
                sems_g[slot],
            )

    def wait_gather(i):
        slot = i % NBUF_G
        for g in range(grps_per_unit):
            pltpu.make_async_copy(
                table_hbm.at[idx_v.at[g]],
                bufs[slot].at[pl.ds(g * GRP, GRP)],
                sems_g[slot],
            ).wait()

    def unit_sq(i):
        u = wid * units_per_w + i
        return u // NQ, lax.rem(u, NQ)

    def wait_write(i):
        s, q = unit_sq(i)
        pltpu.make_async_copy(
            obufs[i % NBUF_O].at[:, pl.ds(0, BQ)],
            out_hbm.at[s, :, pl.ds(q * BQ, BQ)],
            sems_w[i % NBUF_O],
        ).wait()

    for i in range(min(NBUF_G, units_per_w)):
        fire_gather(i)

    for i in range(units_per_w):
        s, q = unit_sq(i)
        buf = bufs[i % NBUF_G]
        obuf = obufs[i % NBUF_O]

        with jax.named_scope("xp_gwait"):
            wait_gather(i)
        if i >= NBUF_O:
            with jax.named_scope("xp_wwait"):
                wait_write(i - NBUF_O)

        pe_s = [pe_v[s, pl.ds(db * NLANE, NLANE)] for db in range(D_MODEL // NLANE)]

        def j_body(j):
            col = jnp.full((NLANE,), j, dtype=jnp.int32)
            for db in range(D_MODEL // NLANE):
                v = buf[j, pl.ds(db * NLANE, NLANE)]
                plsc.store_scatter(obuf, [drow[db], col], v * SCALE + pe_s[db])

        with jax.named_scope("xp_compute"):
            plsc.parallel_loop(0, BQ, unroll=4)(j_body)

        pltpu.async_copy(obuf.at[:, pl.ds(0, BQ)], out_hbm.at[s, :, pl.ds(q * BQ, BQ)],
                         sems_w[i % NBUF_O])
        if i + NBUF_G < units_per_w:
            fire_gather(i + NBUF_G)

    for i in range(units_per_w - NBUF_O, units_per_w):
        wait_write(i)


def kernel(x, table):
    batch, seq = x.shape
    n = batch * seq
    xq = jnp.transpose(x).reshape(NW, n // (NW * GRP), GRP)
    pe = jnp.asarray(_pe_np())

    mesh = plsc.VectorSubcoreMesh(core_axis_name="c", subcore_axis_name="s")
    run = pl.kernel(
        _sc_body,
        mesh=mesh,
        out_type=jax.ShapeDtypeStruct((seq, D_MODEL, batch), jnp.float32),
        scratch_types=[
            pltpu.VMEM((n // (NW * GRP), GRP), jnp.int32),
            pltpu.VMEM((SEQ, D_MODEL), jnp.float32),
            [pltpu.VMEM((BQ, D_MODEL), jnp.float32) for _ in range(NBUF_G)],
            [pltpu.VMEM((D_MODEL, BQP), jnp.float32) for _ in range(NBUF_O)],
            [pltpu.SemaphoreType.DMA for _ in range(NBUF_G)],
            [pltpu.SemaphoreType.DMA for _ in range(NBUF_O)],
        ],
        compiler_params=pltpu.CompilerParams(
            use_tc_tiling_on_sc=False, needs_layout_passes=False),
    )
    out = run(table, xq, pe)
    return jnp.transpose(out, (2, 0, 1))


# speedup vs baseline: 1.3896x; 1.0053x over previous
import jax
import jax.numpy as jnp
import numpy as np
from jax import lax
from jax.experimental import pallas as pl
from jax.experimental.pallas import tpu as pltpu
from jax.experimental.pallas import tpu_sc as plsc

D_MODEL = 64
MAX_LEN = 256
SEQ = 200

NW = 32
GRP = 128
BQ = 256
BQP = BQ + 1
NQ = 4
NLANE = 16
SCALE = float(np.sqrt(np.float32(D_MODEL)))


def _pe_np():
    pos = np.arange(MAX_LEN)[:, np.newaxis]
    i = np.arange(D_MODEL)[np.newaxis, :]
    angle_rates = 1 / np.power(10000, 2 * (i // 2) / np.float32(D_MODEL))
    angle_rads = pos * angle_rates
    pe = np.zeros((MAX_LEN, D_MODEL), dtype=np.float32)
    pe[:, 0::2] = np.sin(angle_rads[:, 0::2])
    pe[:, 1::2] = np.cos(angle_rads[:, 1::2])
    return pe[:SEQ]


NBUF_G = 3
NBUF_O = 2


def _sc_body(table_hbm, idx_hbm, pe_hbm, out_hbm, idx_v, pe_v,
             bufs, obufs, sems_g, sems_w):
    wid = lax.axis_index("s") * 2 + lax.axis_index("c")
    units_per_w = (SEQ * NQ) // NW
    grps_per_unit = BQ // GRP

    q = wid // 8
    s0 = 25 * lax.rem(wid, 8)

    pltpu.sync_copy(
        idx_hbm.at[pl.ds(s0, units_per_w), pl.ds(q * grps_per_unit, grps_per_unit)],
        idx_v)
    pltpu.sync_copy(pe_hbm, pe_v)

    drow = [lax.iota(jnp.int32, NLANE) + db * NLANE for db in range(D_MODEL // NLANE)]

    def fire_gather(i):
        slot = i % NBUF_G
        for g in range(grps_per_unit):
            pltpu.async_copy(
                table_hbm.at[idx_v.at[i, g]],
                bufs[slot].at[pl.ds(g * GRP, GRP)],
                sems_g[slot],
            )

    def wait_gather(i):
        slot = i % NBUF_G
        for g in range(grps_per_unit):
            pltpu.make_async_copy(
                table_hbm.at[idx_v.at[i, g]],
                bufs[slot].at[pl.ds(g * GRP, GRP)],
                sems_g[slot],
            ).wait()

    def unit_sq(i):
        return s0 + i, q

    def wait_write(i):
        s, _ = unit_sq(i)
        pltpu.make_async_copy(
            obufs[i % NBUF_O].at[:, pl.ds(0, BQ)],
            out_hbm.at[s, :, pl.ds(q * BQ, BQ)],
            sems_w[i % NBUF_O],
        ).wait()

    for i in range(min(NBUF_G, units_per_w)):
        fire_gather(i)

    for i in range(units_per_w):
        s, q = unit_sq(i)
        buf = bufs[i % NBUF_G]
        obuf = obufs[i % NBUF_O]

        with jax.named_scope("xp_gwait"):
            wait_gather(i)
        if i >= NBUF_O:
            with jax.named_scope("xp_wwait"):
                wait_write(i - NBUF_O)

        pe_s = [pe_v[s, pl.ds(db * NLANE, NLANE)] for db in range(D_MODEL // NLANE)]

        def j_body(j):
            col = jnp.full((NLANE,), j, dtype=jnp.int32)
            for db in range(D_MODEL // NLANE):
                v = buf[j, pl.ds(db * NLANE, NLANE)]
                plsc.store_scatter(obuf, [drow[db], col], v * SCALE + pe_s[db])

        with jax.named_scope("xp_compute"):
            plsc.parallel_loop(0, BQ, unroll=4)(j_body)

        pltpu.async_copy(obuf.at[:, pl.ds(0, BQ)], out_hbm.at[s, :, pl.ds(q * BQ, BQ)],
                         sems_w[i % NBUF_O])
        if i + NBUF_G < units_per_w:
            fire_gather(i + NBUF_G)

    for i in range(units_per_w - NBUF_O, units_per_w):
        wait_write(i)


def kernel(x, table):
    batch, seq = x.shape
    n = batch * seq
    xq = jnp.transpose(x).reshape(seq, batch // GRP, GRP)
    pe = jnp.asarray(_pe_np())

    mesh = plsc.VectorSubcoreMesh(core_axis_name="c", subcore_axis_name="s")
    run = pl.kernel(
        _sc_body,
        mesh=mesh,
        out_type=jax.ShapeDtypeStruct((seq, D_MODEL, batch), jnp.float32),
        scratch_types=[
            pltpu.VMEM(((SEQ * NQ) // NW, BQ // GRP, GRP), jnp.int32),
            pltpu.VMEM((SEQ, D_MODEL), jnp.float32),
            [pltpu.VMEM((BQ, D_MODEL), jnp.float32) for _ in range(NBUF_G)],
            [pltpu.VMEM((D_MODEL, BQP), jnp.float32) for _ in range(NBUF_O)],
            [pltpu.SemaphoreType.DMA for _ in range(NBUF_G)],
            [pltpu.SemaphoreType.DMA for _ in range(NBUF_O)],
        ],
        compiler_params=pltpu.CompilerParams(
            use_tc_tiling_on_sc=False, needs_layout_passes=False),
    )
    out = run(table, xq, pe)
    return jnp.transpose(out, (2, 0, 1))


# speedup vs baseline: 1.4437x; 1.0390x over previous
import jax
import jax.numpy as jnp
import numpy as np
from jax import lax
from jax.experimental import pallas as pl
from jax.experimental.pallas import tpu as pltpu
from jax.experimental.pallas import tpu_sc as plsc

D_MODEL = 64
MAX_LEN = 256
SEQ = 200

NW = 32
GRP = 128
BQ = 256
NQ = 4
NLANE = 16
UPW = (SEQ * NQ) // NW
SLAB = 32
SCALE = float(np.sqrt(np.float32(D_MODEL)))

NBUF_G = 2
NBUF_O = 2


def _pe_np():
    pos = np.arange(MAX_LEN)[:, np.newaxis]
    i = np.arange(D_MODEL)[np.newaxis, :]
    angle_rates = 1 / np.power(10000, 2 * (i // 2) / np.float32(D_MODEL))
    angle_rads = pos * angle_rates
    pe = np.zeros((MAX_LEN, D_MODEL), dtype=np.float32)
    pe[:, 0::2] = np.sin(angle_rads[:, 0::2])
    pe[:, 1::2] = np.cos(angle_rads[:, 1::2])
    return pe[:SEQ]


def _sc_body(table_hbm, idx_hbm, pe_hbm, out_hbm, idx_v, par_v, pe_v,
             bufs, obufs, sems_g, sems_w):
    wid = lax.axis_index("s") * 2 + lax.axis_index("c")
    grps_per_unit = BQ // GRP

    q = wid // 8
    s0 = UPW * lax.rem(wid, 8)
    s0a = 8 * (s0 // 8)
    off = s0 - s0a

    pltpu.sync_copy(idx_hbm.at[pl.ds(s0a, SLAB), pl.ds(q * BQ, BQ)], idx_v)
    pltpu.sync_copy(pe_hbm.at[pl.ds(s0a, SLAB)], pe_v)

    def prep_body(t):
        r = t // (BQ // NLANE)
        jb = lax.rem(t, BQ // NLANE)
        sl = pl.ds(jb * NLANE, NLANE)
        v = idx_v[r, sl]
        par_v[pl.ds(t * NLANE, NLANE)] = lax.rem(v, 2)
        idx_v[r, sl] = v // 2

    plsc.parallel_loop(0, SLAB * (BQ // NLANE), unroll=4)(prep_body)

    lane = lax.iota(jnp.int32, NLANE)
    drow = [lane + db * NLANE for db in range(D_MODEL // NLANE)]

    def fire_gather(i):
        slot = i % NBUF_G
        for g in range(grps_per_unit):
            pltpu.async_copy(
                table_hbm.at[idx_v.at[off + i, pl.ds(g * GRP, GRP)]],
                bufs[slot].at[pl.ds(g * GRP, GRP)],
                sems_g[slot],
            )

    def wait_gather(i):
        slot = i % NBUF_G
        for g in range(grps_per_unit):
            pltpu.make_async_copy(
                table_hbm.at[idx_v.at[off + i, pl.ds(g * GRP, GRP)]],
                bufs[slot].at[pl.ds(g * GRP, GRP)],
                sems_g[slot],
            ).wait()

    def wait_write(i):
        pltpu.make_async_copy(
            obufs[i % NBUF_O],
            out_hbm.at[s0 + i, :, pl.ds(q * BQ, BQ)],
            sems_w[i % NBUF_O],
        ).wait()

    for i in range(min(NBUF_G, UPW)):
        fire_gather(i)

    for i in range(UPW):
        s = s0 + i
        buf = bufs[i % NBUF_G]
        obuf = obufs[i % NBUF_O]

        wait_gather(i)
        if i >= NBUF_O:
            wait_write(i - NBUF_O)

        parbase = (off + i) * BQ
        pe_s = [pe_v[off + i, pl.ds(db * NLANE, NLANE)]
                for db in range(D_MODEL // NLANE)]

        def j_body(j0):
            jv = lane + j0
            jw = jnp.where(jv >= BQ, jv - BQ, jv)
            par = plsc.load_gather(par_v, [jw + parbase])
            colbase = par * D_MODEL + lane
            for db in range(D_MODEL // NLANE):
                v = plsc.load_gather(buf, [jw, colbase + db * NLANE])
                plsc.store_scatter(obuf, [drow[db], jw], v * SCALE + pe_s[db])

        plsc.parallel_loop(0, BQ, unroll=4)(j_body)

        pltpu.async_copy(obuf, out_hbm.at[s, :, pl.ds(q * BQ, BQ)],
                         sems_w[i % NBUF_O])
        if i + NBUF_G < UPW:
            fire_gather(i + NBUF_G)

    for i in range(UPW - NBUF_O, UPW):
        wait_write(i)


def kernel(x, table):
    batch, seq = x.shape
    vocab = table.shape[0]
    xq = jnp.transpose(x)
    t2 = table.reshape(vocab // 2, 2 * D_MODEL)
    pe = jnp.asarray(_pe_np())

    mesh = plsc.VectorSubcoreMesh(core_axis_name="c", subcore_axis_name="s")
    run = pl.kernel(
        _sc_body,
        mesh=mesh,
        out_type=jax.ShapeDtypeStruct((seq, D_MODEL, batch), jnp.float32),
        scratch_types=[
            pltpu.VMEM((SLAB, BQ), jnp.int32),
            pltpu.VMEM((SLAB * BQ,), jnp.int32),
            pltpu.VMEM((SLAB, D_MODEL), jnp.float32),
            [pltpu.VMEM((BQ, 2 * D_MODEL), jnp.float32) for _ in range(NBUF_G)],
            [pltpu.VMEM((D_MODEL, BQ), jnp.float32) for _ in range(NBUF_O)],
            [pltpu.SemaphoreType.DMA for _ in range(NBUF_G)],
            [pltpu.SemaphoreType.DMA for _ in range(NBUF_O)],
        ],
        compiler_params=pltpu.CompilerParams(
            use_tc_tiling_on_sc=True, needs_layout_passes=False),
    )
    out = run(t2, xq, pe)
    return jnp.transpose(out, (2, 0, 1))


# speedup vs baseline: 1.6858x; 1.1677x over previous
import jax
import jax.numpy as jnp
import numpy as np
from jax import lax
from jax.experimental import pallas as pl
from jax.experimental.pallas import tpu as pltpu
from jax.experimental.pallas import tpu_sc as plsc

D_MODEL = 64
MAX_LEN = 256
SEQ = 200

NW = 32
GRP = 128
BQ = 256
NQ = 4
NLANE = 16
UPW = (SEQ * NQ) // NW
HALF_VOCAB = 500000
SLAB = 32
SCALE = float(np.sqrt(np.float32(D_MODEL)))

NBUF_G = 2
NBUF_O = 2


def _pe_np():
    pos = np.arange(MAX_LEN)[:, np.newaxis]
    i = np.arange(D_MODEL)[np.newaxis, :]
    angle_rates = 1 / np.power(10000, 2 * (i // 2) / np.float32(D_MODEL))
    angle_rads = pos * angle_rates
    pe = np.zeros((MAX_LEN, D_MODEL), dtype=np.float32)
    pe[:, 0::2] = np.sin(angle_rads[:, 0::2])
    pe[:, 1::2] = np.cos(angle_rads[:, 1::2])
    return pe[:SEQ]


def _sc_body(table_hbm, idx_hbm, pe_hbm, out_hbm, idx_v, par_v, pe_v,
             bufs, obufs, sems_g, sems_w):
    wid = lax.axis_index("s") * 2 + lax.axis_index("c")
    grps_per_unit = BQ // GRP

    q = wid // 8
    s0 = UPW * lax.rem(wid, 8)
    s0a = 8 * (s0 // 8)
    off = s0 - s0a

    pltpu.sync_copy(idx_hbm.at[pl.ds(s0a, SLAB), pl.ds(q * BQ, BQ)], idx_v)
    pltpu.sync_copy(pe_hbm.at[pl.ds(s0a, SLAB)], pe_v)

    def prep_body(t):
        r = t // (BQ // NLANE)
        jb = lax.rem(t, BQ // NLANE)
        sl = pl.ds(jb * NLANE, NLANE)
        v = idx_v[r, sl]
        hi = jnp.where(v >= HALF_VOCAB, 1, 0).astype(jnp.int32)
        par_v[pl.ds(t * NLANE, NLANE)] = hi
        idx_v[r, sl] = v - hi * HALF_VOCAB

    plsc.parallel_loop(0, SLAB * (BQ // NLANE), unroll=4)(prep_body)

    lane = lax.iota(jnp.int32, NLANE)
    drow = [lane + db * NLANE for db in range(D_MODEL // NLANE)]

    def fire_gather(i):
        slot = i % NBUF_G
        for g in range(grps_per_unit):
            pltpu.async_copy(
                table_hbm.at[idx_v.at[off + i, pl.ds(g * GRP, GRP)]],
                bufs[slot].at[pl.ds(g * GRP, GRP)],
                sems_g[slot],
            )

    def wait_gather(i):
        slot = i % NBUF_G
        for g in range(grps_per_unit):
            pltpu.make_async_copy(
                table_hbm.at[idx_v.at[off + i, pl.ds(g * GRP, GRP)]],
                bufs[slot].at[pl.ds(g * GRP, GRP)],
                sems_g[slot],
            ).wait()

    def wait_write(i):
        pltpu.make_async_copy(
            obufs[i % NBUF_O],
            out_hbm.at[s0 + i, :, pl.ds(q * BQ, BQ)],
            sems_w[i % NBUF_O],
        ).wait()

    for i in range(min(NBUF_G, UPW)):
        fire_gather(i)

    for i in range(UPW):
        s = s0 + i
        buf = bufs[i % NBUF_G]
        obuf = obufs[i % NBUF_O]

        wait_gather(i)
        if i >= NBUF_O:
            wait_write(i - NBUF_O)

        parbase = (off + i) * BQ
        pe_s = [pe_v[off + i, pl.ds(db * NLANE, NLANE)]
                for db in range(D_MODEL // NLANE)]

        def j_body(j0):
            jv = lane + j0
            jw = jnp.where(jv >= BQ, jv - BQ, jv)
            par = plsc.load_gather(par_v, [jw + parbase])
            colbase = 2 * lane + par
            for db in range(D_MODEL // NLANE):
                v = plsc.load_gather(buf, [jw, colbase + db * (2 * NLANE)])
                plsc.store_scatter(obuf, [drow[db], jw], v * SCALE + pe_s[db])

        plsc.parallel_loop(0, BQ, unroll=4)(j_body)

        pltpu.async_copy(obuf, out_hbm.at[s, :, pl.ds(q * BQ, BQ)],
                         sems_w[i % NBUF_O])
        if i + NBUF_G < UPW:
            fire_gather(i + NBUF_G)

    for i in range(UPW - NBUF_O, UPW):
        wait_write(i)


def kernel(x, table):
    batch, seq = x.shape
    vocab = table.shape[0]
    xq = jnp.transpose(x)
    t2 = jnp.transpose(jnp.transpose(table).reshape(2 * D_MODEL, vocab // 2))
    pe = jnp.asarray(_pe_np())

    mesh = plsc.VectorSubcoreMesh(core_axis_name="c", subcore_axis_name="s")
    run = pl.kernel(
        _sc_body,
        mesh=mesh,
        out_type=jax.ShapeDtypeStruct((seq, D_MODEL, batch), jnp.float32),
        scratch_types=[
            pltpu.VMEM((SLAB, BQ), jnp.int32),
            pltpu.VMEM((SLAB * BQ,), jnp.int32),
            pltpu.VMEM((SLAB, D_MODEL), jnp.float32),
            [pltpu.VMEM((BQ, 2 * D_MODEL), jnp.float32) for _ in range(NBUF_G)],
            [pltpu.VMEM((D_MODEL, BQ), jnp.float32) for _ in range(NBUF_O)],
            [pltpu.SemaphoreType.DMA for _ in range(NBUF_G)],
            [pltpu.SemaphoreType.DMA for _ in range(NBUF_O)],
        ],
        compiler_params=pltpu.CompilerParams(
            use_tc_tiling_on_sc=True, needs_layout_passes=False),
    )
    out = run(t2, xq, pe)
    return jnp.transpose(out, (2, 0, 1))
